# baseline (speedup 1.0000x reference)
import functools

import jax
import jax.numpy as jnp
from jax.experimental import pallas as pl


def _matmul_body(x_ref, w_ref, o_ref):
    o_ref[...] = jnp.dot(x_ref[...], w_ref[...],
                         preferred_element_type=jnp.float32)


def _matmul(x, w):
    npad, f = x.shape
    c = w.shape[1]
    nb = npad // 256
    return pl.pallas_call(
        _matmul_body,
        grid=(nb,),
        in_specs=[
            pl.BlockSpec((256, f), lambda i: (i, 0)),
            pl.BlockSpec((f, c), lambda i: (0, 0)),
        ],
        out_specs=pl.BlockSpec((256, c), lambda i: (i, 0)),
        out_shape=jax.ShapeDtypeStruct((npad, c), jnp.float32),
    )(x, w)


def _combine_body(relu, agg_ref, deg_ref, selfb_ref, o_ref):
    deg = jnp.maximum(deg_ref[0, 0, :], 1.0)
    out = agg_ref[...] / deg[:, None] + selfb_ref[...]
    if relu:
        out = jnp.maximum(out, 0.0)
    o_ref[...] = out


def _combine(agg, deg, selfb, relu):
    npad, h = agg.shape
    nb = npad // 128
    deg2 = deg.reshape(nb, 1, 128)
    return pl.pallas_call(
        functools.partial(_combine_body, relu),
        grid=(nb,),
        in_specs=[
            pl.BlockSpec((128, h), lambda i: (i, 0)),
            pl.BlockSpec((1, 1, 128), lambda i: (i, 0, 0)),
            pl.BlockSpec((128, h), lambda i: (i, 0)),
        ],
        out_specs=pl.BlockSpec((128, h), lambda i: (i, 0)),
        out_shape=jax.ShapeDtypeStruct((npad, h), jnp.float32),
    )(agg, deg2, selfb)


def _layer(x, d, key_idx, deg, wcat, wself, bias, relu):
    n, f = x.shape
    h = wself.shape[1]
    r = wcat.shape[1] // h
    npad = ((n + 255) // 256) * 256
    xp = jnp.pad(x, ((0, npad - n), (0, 0)))
    y = _matmul(xp, wcat)[:n].reshape(n * r, h)
    selfb = _matmul(xp, wself) + bias
    msgs = y[key_idx]
    agg = jnp.zeros((n, h), jnp.float32).at[d].add(msgs)
    pad = npad - n
    out = _combine(
        jnp.pad(agg, ((0, pad), (0, 0))),
        jnp.pad(deg, ((0, pad),)),
        selfb,
        relu,
    )
    return out[:n]


def kernel(edge_index, edge_type, src, dst, rel, node_features, num_nodes,
           bases0, coeffs0, wself0, bias0,
           bases1, coeffs1, wself1, bias1,
           rel_emb, wcls, bcls):
    n, f = node_features.shape
    r = coeffs0.shape[0]
    d = edge_index[1]
    key_idx = edge_index[0] * r + edge_type
    deg = jnp.zeros((n,), jnp.float32).at[d].add(1.0)

    w0 = jnp.einsum('rb,bfh->frh', coeffs0, bases0).reshape(f, -1)
    h1 = _layer(node_features, d, key_idx, deg, w0, wself0, bias0, relu=True)
    w1 = jnp.einsum('rb,bfh->frh', coeffs1, bases1).reshape(h1.shape[1], -1)
    emb = _layer(h1, d, key_idx, deg, w1, wself1, bias1, relu=False)

    link_scores = jnp.sum(emb[src] * rel_emb[rel] * emb[dst], axis=-1)
    node_logits = emb @ wcls + bcls
    return (link_scores, node_logits)


# speedup vs baseline: 1.8408x; 1.8408x over previous
import functools

import jax
import jax.numpy as jnp
from jax import lax
from jax.experimental import pallas as pl
from jax.experimental.pallas import tpu as pltpu
from jax.experimental.pallas import tpu_sc as plsc

_N = 10000
_NPAD = 10240
_DUMMY = 10100
_EPT = 10112
_NCH = _EPT // 128
_EP = _EPT * 16


def _matmul_body(x_ref, w_ref, o_ref):
    o_ref[...] = jnp.dot(x_ref[...], w_ref[...],
                         preferred_element_type=jnp.float32)


def _matmul(x, w):
    npad, f = x.shape
    c = w.shape[1]
    nb = npad // 256
    return pl.pallas_call(
        _matmul_body,
        grid=(nb,),
        in_specs=[
            pl.BlockSpec((256, f), lambda i: (i, 0)),
            pl.BlockSpec((f, c), lambda i: (0, 0)),
        ],
        out_specs=pl.BlockSpec((256, c), lambda i: (i, 0)),
        out_shape=jax.ShapeDtypeStruct((npad, c), jnp.float32),
    )(x, w)


def _sc_aggregate(ytab, gidx, dste):
    mesh = plsc.VectorSubcoreMesh(core_axis_name="c", subcore_axis_name="s")
    zrows = jnp.zeros((640, 128), jnp.float32)

    @functools.partial(
        pl.kernel, mesh=mesh,
        out_type=jax.ShapeDtypeStruct((2 * _NPAD, 128), jnp.float32),
        scratch_types=[
            pltpu.VMEM((128,), jnp.int32),
            pltpu.VMEM((128,), jnp.int32),
            pltpu.VMEM((128, 128), jnp.float32),
            pltpu.SemaphoreType.DMA,
            pltpu.VMEM_SHARED((_NPAD, 128), jnp.float32),
        ],
    )
    def k(gidx_h, dst_h, zrows_h, ytab_h, agg_o,
          idx_v, dst_v, rows_v, sem, agg_s):
        c = lax.axis_index("c")
        s = lax.axis_index("s")
        rbase = s * 640
        pltpu.sync_copy(zrows_h, agg_s.at[pl.ds(rbase, 640)])
        plsc.subcore_barrier()

        ebase = s * _EPT
        gbase = c * _EP + ebase

        def chunk(j, carry):
            go = pl.multiple_of(gbase + j * 128, 128)
            eo = pl.multiple_of(ebase + j * 128, 128)
            pltpu.sync_copy(gidx_h.at[pl.ds(go, 128)], idx_v)
            pltpu.sync_copy(dst_h.at[pl.ds(eo, 128)], dst_v)
            pltpu.async_copy(ytab_h.at[idx_v], rows_v, sem).wait()
            pltpu.sync_copy(rows_v, agg_s.at[dst_v], add=True)
            return carry

        lax.fori_loop(0, _NCH, chunk, 0)
        plsc.subcore_barrier()

        obase = pl.multiple_of(c * _NPAD + rbase, 128)
        pltpu.sync_copy(agg_s.at[pl.ds(rbase, 640)], agg_o.at[pl.ds(obase, 640)])

    return k(gidx, dste, zrows, ytab)


_EPT2 = 5120
_NCH2 = _EPT2 // 128
_EP2 = _EPT2 * 32


def _sc_degree(dste2):
    mesh = plsc.VectorSubcoreMesh(core_axis_name="c", subcore_axis_name="s")
    zrows = jnp.zeros((640, 128), jnp.float32)
    ones = jnp.ones((128, 128), jnp.float32)

    @functools.partial(
        pl.kernel, mesh=mesh,
        out_type=jax.ShapeDtypeStruct((2 * _NPAD, 128), jnp.float32),
        scratch_types=[
            pltpu.VMEM((128,), jnp.int32),
            pltpu.VMEM((128, 128), jnp.float32),
            pltpu.VMEM_SHARED((_NPAD, 128), jnp.float32),
        ],
    )
    def k(dst_h, zrows_h, ones_h, deg_o, dst_v, ones_v, deg_s):
        c = lax.axis_index("c")
        s = lax.axis_index("s")
        rbase = s * 640
        pltpu.sync_copy(zrows_h, deg_s.at[pl.ds(rbase, 640)])
        pltpu.sync_copy(ones_h, ones_v)
        plsc.subcore_barrier()

        ebase = (s * 2 + c) * _EPT2

        def chunk(j, carry):
            eo = pl.multiple_of(ebase + j * 128, 128)
            pltpu.sync_copy(dst_h.at[pl.ds(eo, 128)], dst_v)
            pltpu.sync_copy(ones_v, deg_s.at[dst_v], add=True)
            return carry

        lax.fori_loop(0, _NCH2, chunk, 0)
        plsc.subcore_barrier()

        obase = pl.multiple_of(c * _NPAD + rbase, 128)
        pltpu.sync_copy(deg_s.at[pl.ds(rbase, 640)], deg_o.at[pl.ds(obase, 640)])

    return k(dste2, zrows, ones)


def _combine_body(relu, agg_ref, deg_ref, selfb_ref, o_ref):
    deg = jnp.maximum(deg_ref[0, 0, :], 1.0)
    out = agg_ref[0] / deg[:, None] + selfb_ref[...]
    if relu:
        out = jnp.maximum(out, 0.0)
    o_ref[...] = out


def _combine(aggs, deg2, selfb, relu):
    nb = _NPAD // 128
    return pl.pallas_call(
        functools.partial(_combine_body, relu),
        grid=(nb, 2),
        in_specs=[
            pl.BlockSpec((1, 128, 128), lambda i, c: (c, i, 0)),
            pl.BlockSpec((1, 1, 128), lambda i, c: (i, 0, 0)),
            pl.BlockSpec((128, 128), lambda i, c: (i, c)),
        ],
        out_specs=pl.BlockSpec((128, 128), lambda i, c: (i, c)),
        out_shape=jax.ShapeDtypeStruct((_NPAD, 256), jnp.float32),
    )(aggs, deg2, selfb)


def _layer(xp, gidx, dste, deg2, wcat, wself, bias, relu):
    y = _matmul(xp, wcat)
    selfb = _matmul(xp, wself) + bias
    ytab = y.reshape(_NPAD * (y.shape[1] // 128), 128)
    agg = _sc_aggregate(ytab, gidx, dste)
    return _combine(agg.reshape(2, _NPAD, 128), deg2, selfb, relu)


def kernel(edge_index, edge_type, src, dst, rel, node_features, num_nodes,
           bases0, coeffs0, wself0, bias0,
           bases1, coeffs1, wself1, bias1,
           rel_emb, wcls, bcls):
    n, f = node_features.shape
    r = coeffs0.shape[0]
    h = wself0.shape[1]

    g0 = edge_index[0] * (2 * r) + edge_type * 2
    pad = _EP - g0.shape[0]
    g0p = jnp.pad(g0, (0, pad))
    gidx = jnp.concatenate([g0p, g0p + 1]).astype(jnp.int32)
    dste = jnp.pad(edge_index[1], (0, pad),
                   constant_values=_DUMMY).astype(jnp.int32)
    dste2 = jnp.pad(edge_index[1], (0, _EP2 - g0.shape[0]),
                    constant_values=_DUMMY).astype(jnp.int32)

    degp = _sc_degree(dste2)
    deg2 = (degp[:_NPAD, 0] + degp[_NPAD:, 0]).reshape(_NPAD // 128, 1, 128)

    xp = jnp.pad(node_features, ((0, _NPAD - n), (0, 0)))
    w0 = jnp.einsum('rb,bfh->frh', coeffs0, bases0).reshape(f, r * h)
    h1 = _layer(xp, gidx, dste, deg2, w0, wself0, bias0, relu=True)
    w1 = jnp.einsum('rb,bfh->frh', coeffs1, bases1).reshape(h, r * h)
    emb_p = _layer(h1, gidx, dste, deg2, w1, wself1, bias1, relu=False)
    emb = emb_p[:n]

    link_scores = jnp.sum(emb[src] * rel_emb[rel] * emb[dst], axis=-1)
    node_logits = emb @ wcls + bcls
    return (link_scores, node_logits)
